# initial kernel scaffold (unmeasured)
import jax
import jax.numpy as jnp
from jax import lax
from jax.experimental import pallas as pl
from jax.experimental.pallas import tpu as pltpu


def kernel(
    x,
):
    def body(*refs):
        pass

    out_shape = jax.ShapeDtypeStruct(..., jnp.float32)
    return pl.pallas_call(body, out_shape=out_shape)(...)


# baseline (device time: 100027 ns/iter reference)
import jax
import jax.numpy as jnp
from jax import lax
from jax.experimental import pallas as pl
from jax.experimental.pallas import tpu as pltpu

N_DEV = 8


def kernel(x):
    _, m, n = x.shape
    ch = n // N_DEV

    def body(x_ref, out_ref, comm_ref, send_sems, recv_sems):
        my = lax.axis_index("i")
        left = (my + N_DEV - 1) % N_DEV
        right = (my + 1) % N_DEV

        barrier_sem = pltpu.get_barrier_semaphore()
        for nbr in (left, right):
            pl.semaphore_signal(
                barrier_sem, inc=1,
                device_id=(nbr,), device_id_type=pl.DeviceIdType.MESH,
            )
        pl.semaphore_wait(barrier_sem, 2)

        c0 = (my + N_DEV - 1) % N_DEV
        comm_ref[0, :, :] = x_ref[0, :, pl.ds(c0 * ch, ch)]

        for s in range(N_DEV - 1):
            rdma = pltpu.make_async_remote_copy(
                src_ref=comm_ref.at[s],
                dst_ref=comm_ref.at[s + 1],
                send_sem=send_sems.at[s],
                recv_sem=recv_sems.at[s],
                device_id=(right,),
                device_id_type=pl.DeviceIdType.MESH,
            )
            rdma.start()
            rdma.wait()

            c = (my + 2 * N_DEV - 2 - s) % N_DEV
            chunk = x_ref[0, :, pl.ds(c * ch, ch)]
            if s < N_DEV - 2:
                comm_ref[s + 1, :, :] = comm_ref[s + 1, :, :] + chunk
            else:
                out_ref[:, :] = comm_ref[s + 1, :, :] + chunk

    return pl.pallas_call(
        body,
        out_shape=jax.ShapeDtypeStruct((m, ch), jnp.float32),
        in_specs=[pl.BlockSpec(memory_space=pltpu.VMEM)],
        out_specs=pl.BlockSpec(memory_space=pltpu.VMEM),
        scratch_shapes=[
            pltpu.VMEM((N_DEV, m, ch), jnp.float32),
            pltpu.SemaphoreType.DMA((N_DEV - 1,)),
            pltpu.SemaphoreType.DMA((N_DEV - 1,)),
        ],
        compiler_params=pltpu.CompilerParams(collective_id=0),
    )(x)


# device time: 63749 ns/iter; 1.5691x vs baseline; 1.5691x over previous
import jax
import jax.numpy as jnp
from jax import lax
from jax.experimental import pallas as pl
from jax.experimental.pallas import tpu as pltpu

N_DEV = 8


def kernel(x):
    _, m, n = x.shape
    ch = n // N_DEV
    half = m // 2

    def body(x_ref, out_ref, comm_r, comm_l,
             send_r, recv_r, send_l, recv_l):
        my = lax.axis_index("i")
        left = (my + N_DEV - 1) % N_DEV
        right = (my + 1) % N_DEV

        barrier_sem = pltpu.get_barrier_semaphore()
        for nbr in (left, right):
            pl.semaphore_signal(
                barrier_sem, inc=1,
                device_id=(nbr,), device_id_type=pl.DeviceIdType.MESH,
            )
        pl.semaphore_wait(barrier_sem, 2)

        cr = (my + N_DEV - 1) % N_DEV
        cl = (my + 1) % N_DEV
        comm_r[0, :, :] = x_ref[0, 0:half, pl.ds(cr * ch, ch)]
        comm_l[0, :, :] = x_ref[0, half:m, pl.ds(cl * ch, ch)]

        for s in range(N_DEV - 1):
            rdma_r = pltpu.make_async_remote_copy(
                src_ref=comm_r.at[s],
                dst_ref=comm_r.at[s + 1],
                send_sem=send_r.at[s],
                recv_sem=recv_r.at[s],
                device_id=(right,),
                device_id_type=pl.DeviceIdType.MESH,
            )
            rdma_l = pltpu.make_async_remote_copy(
                src_ref=comm_l.at[s],
                dst_ref=comm_l.at[s + 1],
                send_sem=send_l.at[s],
                recv_sem=recv_l.at[s],
                device_id=(left,),
                device_id_type=pl.DeviceIdType.MESH,
            )
            rdma_r.start()
            rdma_l.start()
            rdma_r.wait_recv()
            rdma_l.wait_recv()

            c_r = (my + 2 * N_DEV - 2 - s) % N_DEV
            c_l = (my + 2 + s) % N_DEV
            top = x_ref[0, 0:half, pl.ds(c_r * ch, ch)]
            bot = x_ref[0, half:m, pl.ds(c_l * ch, ch)]
            if s < N_DEV - 2:
                comm_r[s + 1, :, :] = comm_r[s + 1, :, :] + top
                comm_l[s + 1, :, :] = comm_l[s + 1, :, :] + bot
            else:
                out_ref[0:half, :] = comm_r[s + 1, :, :] + top
                out_ref[half:m, :] = comm_l[s + 1, :, :] + bot

            rdma_r.wait_send()
            rdma_l.wait_send()

    return pl.pallas_call(
        body,
        out_shape=jax.ShapeDtypeStruct((m, ch), jnp.float32),
        in_specs=[pl.BlockSpec(memory_space=pltpu.VMEM)],
        out_specs=pl.BlockSpec(memory_space=pltpu.VMEM),
        scratch_shapes=[
            pltpu.VMEM((N_DEV, half, ch), jnp.float32),
            pltpu.VMEM((N_DEV, half, ch), jnp.float32),
            pltpu.SemaphoreType.DMA((N_DEV - 1,)),
            pltpu.SemaphoreType.DMA((N_DEV - 1,)),
            pltpu.SemaphoreType.DMA((N_DEV - 1,)),
            pltpu.SemaphoreType.DMA((N_DEV - 1,)),
        ],
        compiler_params=pltpu.CompilerParams(collective_id=0),
    )(x)


# device time: 50467 ns/iter; 1.9820x vs baseline; 1.2632x over previous
import jax
import jax.numpy as jnp
from jax import lax
from jax.experimental import pallas as pl
from jax.experimental.pallas import tpu as pltpu

N_DEV = 8
SUB = 2


def kernel(x):
    _, m, n = x.shape
    ch = n // N_DEV
    half = m // 2
    sub = half // SUB

    def body(x_ref, out_ref, comm_r, comm_l,
             send_r, recv_r, send_l, recv_l):
        my = lax.axis_index("i")
        left = (my + N_DEV - 1) % N_DEV
        right = (my + 1) % N_DEV

        barrier_sem = pltpu.get_barrier_semaphore()
        for nbr in (left, right):
            pl.semaphore_signal(
                barrier_sem, inc=1,
                device_id=(nbr,), device_id_type=pl.DeviceIdType.MESH,
            )
        pl.semaphore_wait(barrier_sem, 2)

        def rdma(dir_comm, dir_send, dir_recv, s, t, dst):
            return pltpu.make_async_remote_copy(
                src_ref=dir_comm.at[s, t],
                dst_ref=dir_comm.at[s + 1, t],
                send_sem=dir_send.at[s, t],
                recv_sem=dir_recv.at[s, t],
                device_id=(dst,),
                device_id_type=pl.DeviceIdType.MESH,
            )

        def row(t):
            return pl.ds(t * sub, sub)

        def c_right(s):
            return (my + 2 * N_DEV - 2 - s) % N_DEV

        def c_left(s):
            return (my + 2 + s) % N_DEV

        for t in range(SUB):
            comm_r[0, t, :, :] = x_ref[0, t * sub:(t + 1) * sub,
                                       pl.ds(c_right(-1) * ch, ch)]
            comm_l[0, t, :, :] = x_ref[0, half + t * sub:half + (t + 1) * sub,
                                       pl.ds(c_left(-1) * ch, ch)]
            rdma(comm_r, send_r, recv_r, 0, t, right).start()
            rdma(comm_l, send_l, recv_l, 0, t, left).start()

        for s in range(N_DEV - 1):
            last = s == N_DEV - 2
            for t in range(SUB):
                r = rdma(comm_r, send_r, recv_r, s, t, right)
                l = rdma(comm_l, send_l, recv_l, s, t, left)
                r.wait_recv()
                top = x_ref[0, t * sub:(t + 1) * sub,
                            pl.ds(c_right(s) * ch, ch)]
                if last:
                    out_ref[row(t), :] = comm_r[s + 1, t, :, :] + top
                else:
                    comm_r[s + 1, t, :, :] = comm_r[s + 1, t, :, :] + top
                    rdma(comm_r, send_r, recv_r, s + 1, t, right).start()
                l.wait_recv()
                bot = x_ref[0, half + t * sub:half + (t + 1) * sub,
                            pl.ds(c_left(s) * ch, ch)]
                if last:
                    out_ref[pl.ds(half + t * sub, sub), :] = (
                        comm_l[s + 1, t, :, :] + bot
                    )
                else:
                    comm_l[s + 1, t, :, :] = comm_l[s + 1, t, :, :] + bot
                    rdma(comm_l, send_l, recv_l, s + 1, t, left).start()

        for s in range(N_DEV - 1):
            for t in range(SUB):
                rdma(comm_r, send_r, recv_r, s, t, right).wait_send()
                rdma(comm_l, send_l, recv_l, s, t, left).wait_send()

    return pl.pallas_call(
        body,
        out_shape=jax.ShapeDtypeStruct((m, ch), jnp.float32),
        in_specs=[pl.BlockSpec(memory_space=pltpu.VMEM)],
        out_specs=pl.BlockSpec(memory_space=pltpu.VMEM),
        scratch_shapes=[
            pltpu.VMEM((N_DEV, SUB, sub, ch), jnp.float32),
            pltpu.VMEM((N_DEV, SUB, sub, ch), jnp.float32),
            pltpu.SemaphoreType.DMA((N_DEV - 1, SUB)),
            pltpu.SemaphoreType.DMA((N_DEV - 1, SUB)),
            pltpu.SemaphoreType.DMA((N_DEV - 1, SUB)),
            pltpu.SemaphoreType.DMA((N_DEV - 1, SUB)),
        ],
        compiler_params=pltpu.CompilerParams(collective_id=0),
    )(x)
